# Initial kernel scaffold; baseline (speedup 1.0000x reference)
import jax
import jax.numpy as jnp
from jax.experimental import pallas as pl


def kernel(x, scale):
    raise NotImplementedError("write your pallas kernel here")


# speedup vs baseline: 3759.1000x; 3759.1000x over previous
import jax
import jax.numpy as jnp
from jax.experimental import pallas as pl
from jax.experimental.pallas import tpu as pltpu

_ROWS = 4096
_COLS = 8192
_BLOCK_ROWS = 256


def _qhash_body(scale_ref, x_ref, o_ref):
    scale = scale_ref[0]
    x = x_ref[...]
    q = jnp.clip(jnp.round(x * 16.0), -128.0, 127.0) * (1.0 / 16.0)
    e = jnp.clip(jnp.round(jnp.exp(q * scale) * 64.0), -128.0, 127.0) * (1.0 / 64.0)
    srow = jnp.sum(e, axis=-1, keepdims=True)
    nf = jnp.clip(jnp.round(jnp.floor(srow / e)), -1024.0, 1023.0)
    nf = jnp.where(e == 0.0, 1023.0, nf)
    o_ref[...] = jnp.clip(jnp.round(128.0 / nf), -128.0, 127.0) * (1.0 / 128.0)


def kernel(x, scale):
    orig_shape = x.shape
    xf = x.reshape(_ROWS, _COLS)
    out = pl.pallas_call(
        _qhash_body,
        grid=(_ROWS // _BLOCK_ROWS,),
        in_specs=[
            pl.BlockSpec(memory_space=pltpu.SMEM),
            pl.BlockSpec((_BLOCK_ROWS, _COLS), lambda i: (i, 0)),
        ],
        out_specs=pl.BlockSpec((_BLOCK_ROWS, _COLS), lambda i: (i, 0)),
        out_shape=jax.ShapeDtypeStruct((_ROWS, _COLS), jnp.float32),
    )(scale.reshape(1), xf)
    return out.reshape(orig_shape)
